# baseline (speedup 1.0000x reference)
import functools
import math

import jax
import jax.numpy as jnp
import numpy as np
from jax import lax
from jax.experimental import pallas as pl
from jax.experimental.pallas import tpu as pltpu
from jax.experimental.pallas import tpu_sc as plsc

DM = 1024
H = 16
P = 4
D = DM // H
HP = H * P
L_SEQ = 2048
B_SZ = 2

BLKA = 512
CH = 64
ROWS = CH * P
NCHUNK = L_SEQ // CH

_NC = 2
_NS = 16

_p32 = np.arange(32).reshape(2, 16).T.reshape(-1)
_p64 = np.concatenate([_p32, _p32 + 32])
_PERM = np.concatenate([_p64 + 64 * h for h in range(H)])


def _proj_body(xq, xkv, wq, bq, wk, bk, wv, bv, woff, boff, waw, baw,
               q_o, k_o, v_o, g0_o, w1_o, lg_o):
    i = pl.program_id(0)
    x = xq[...]
    y = xkv[...]

    def mm(a, w):
        return lax.dot_general(a, w, (((1,), (1,)), ((), ())),
                               preferred_element_type=jnp.float32)

    q_o[...] = (mm(x, wq[...]) + bq[...]) * (1.0 / math.sqrt(D))
    k_o[...] = (mm(y, wk[...]) + bk[...]).astype(jnp.bfloat16)
    v_o[...] = (mm(y, wv[...]) + bv[...]).astype(jnp.bfloat16)
    off = mm(x, woff[...]) + boff[...]
    lg_o[...] = mm(x, waw[...]) + baw[...]

    rows = i * BLKA + lax.broadcasted_iota(jnp.int32, (BLKA, 1), 0)
    lpos = lax.rem(rows, L_SEQ)
    bidx = rows // L_SEQ
    idx_f = jnp.clip(lpos.astype(jnp.float32) + off, 0.0, float(L_SEQ - 1))
    i0f = jnp.floor(idx_f)
    w1_o[...] = idx_f - i0f
    i0 = i0f.astype(jnp.int32)
    hcol = lax.broadcasted_iota(jnp.int32, (1, HP), 1) // P
    bhoff = (bidx * H + hcol) * L_SEQ
    g0_o[...] = bhoff + i0


def _proj_call(xq, xkv, Wq, bq, Wk, bk, Wv, bv, Woff, boff, Waw, baw):
    n = xq.shape[0]
    grid = (n // BLKA,)
    row_spec = pl.BlockSpec((BLKA, DM), lambda i: (i, 0))
    hp_spec = pl.BlockSpec((BLKA, HP), lambda i: (i, 0))
    full = lambda shape: pl.BlockSpec(shape, lambda i: tuple(0 for _ in shape))
    return pl.pallas_call(
        _proj_body,
        grid=grid,
        in_specs=[
            row_spec, row_spec,
            full((DM, DM)), full((1, DM)),
            full((DM, DM)), full((1, DM)),
            full((DM, DM)), full((1, DM)),
            full((HP, DM)), full((1, HP)),
            full((HP, DM)), full((1, HP)),
        ],
        out_specs=[row_spec, row_spec, row_spec,
                   hp_spec, hp_spec, hp_spec],
        out_shape=[
            jax.ShapeDtypeStruct((n, DM), jnp.float32),
            jax.ShapeDtypeStruct((n, DM), jnp.bfloat16),
            jax.ShapeDtypeStruct((n, DM), jnp.bfloat16),
            jax.ShapeDtypeStruct((n, HP), jnp.int32),
            jax.ShapeDtypeStruct((n, HP), jnp.float32),
            jax.ShapeDtypeStruct((n, HP), jnp.float32),
        ],
    )(xq, xkv, Wq, bq.reshape(1, DM), Wk, bk.reshape(1, DM),
      Wv, bv.reshape(1, DM), Woff, boff.reshape(1, HP),
      Waw, baw.reshape(1, HP))


def _outproj_body(x_ref, w_ref, b_ref, o_ref):
    o_ref[...] = lax.dot_general(
        x_ref[...], w_ref[...], (((1,), (1,)), ((), ())),
        preferred_element_type=jnp.float32) + b_ref[...]


def _outproj_call(x, Wout, bout):
    n = x.shape[0]
    row_spec = pl.BlockSpec((BLKA, DM), lambda i: (i, 0))
    return pl.pallas_call(
        _outproj_body,
        grid=(n // BLKA,),
        in_specs=[row_spec,
                  pl.BlockSpec((DM, DM), lambda i: (0, 0)),
                  pl.BlockSpec((1, DM), lambda i: (0, 0))],
        out_specs=row_spec,
        out_shape=jax.ShapeDtypeStruct((n, DM), jnp.float32),
    )(x, Wout, bout.reshape(1, DM))


def _sc_attend_body(kv_hbm, q_hbm, idx_hbm, w1_hbm, lg_hbm, out_hbm,
                    idx_a, idx_b, kv_a, kv_b, q_a, q_b, w1_a, w1_b,
                    lg_a, lg_b, out_v, sem_a, sem_b):
    cid = lax.axis_index("c")
    sid = lax.axis_index("s")
    bh = sid * _NC + cid
    b = bh // H
    h = lax.rem(bh, H)

    NGATHER = ROWS // 128

    def fire(c, idx_v, kv_v, q_v, w1_v, lg_v, sem):
        pltpu.sync_copy(idx_hbm.at[bh, pl.ds(c * ROWS, ROWS)], idx_v)
        for j in range(NGATHER):
            pltpu.async_copy(kv_hbm.at[idx_v.at[pl.ds(j * 128, 128)]],
                             kv_v.at[pl.ds(j * 128, 128)], sem)
        pltpu.sync_copy(q_hbm.at[b, pl.ds(c * CH, CH), h], q_v)
        pltpu.sync_copy(w1_hbm.at[bh, pl.ds(c * CH * P, CH * P)], w1_v)
        pltpu.sync_copy(lg_hbm.at[bh, pl.ds(c * CH * P, CH * P)], lg_v)

    def drain(idx_v, kv_v, sem):
        for j in range(NGATHER):
            pltpu.make_async_copy(
                kv_hbm.at[idx_v.at[pl.ds(j * 128, 128)]],
                kv_v.at[pl.ds(j * 128, 128)], sem).wait()

    def interp_pair(kv_v, r, off0, off1, w1p):
        x0 = plsc.bitcast(kv_v[r, pl.ds(off0, 16)], jnp.bfloat16)
        x1 = plsc.bitcast(kv_v[r, pl.ds(off1, 16)], jnp.bfloat16)
        e0, o0 = plsc.unpack(x0, format=plsc.PackFormat.INTERLEAVED)
        e1, o1 = plsc.unpack(x1, format=plsc.PackFormat.INTERLEAVED)
        ee = e0 + w1p * (e1 - e0)
        oo = o0 + w1p * (o1 - o0)
        return ee, oo

    def comp(c, kv_v, q_v, w1_v, lg_v):
        def q_body(g, carry2):
            wvec = w1_v[pl.ds(g * 16, 16)]
            lvec = lg_v[pl.ds(g * 16, 16)]
            for qq in range(4):
                i = g * 4 + qq
                qi = [q_v[i, pl.ds(16 * t, 16)] for t in range(4)]
                base = i * P
                w1ps = []
                scrs = []
                for p in range(P):
                    w1p = jnp.full((16,), wvec[4 * qq + p])
                    lgp = lvec[4 * qq + p]
                    w1ps.append(w1p)
                    r = base + p
                    ke0, ko0 = interp_pair(kv_v, r, 0, 64, w1p)
                    ke1, ko1 = interp_pair(kv_v, r, 16, 80, w1p)
                    acc = ((qi[0] * ke0 + qi[1] * ko0)
                           + (qi[2] * ke1 + qi[3] * ko1))
                    scrs.append(lgp + jnp.sum(acc))
                m = jnp.maximum(jnp.maximum(scrs[0], scrs[1]),
                                jnp.maximum(scrs[2], scrs[3]))
                es = [jnp.exp(jnp.full((16,), s - m)) for s in scrs]
                den = (es[0] + es[1]) + (es[2] + es[3])
                wgt = [e / den for e in es]
                outt = [jnp.zeros((16,), jnp.float32) for _ in range(4)]
                for p in range(P):
                    r = base + p
                    for u in range(2):
                        ve, vo = interp_pair(kv_v, r, 32 + 16 * u,
                                             96 + 16 * u, w1ps[p])
                        outt[2 * u] = outt[2 * u] + wgt[p] * ve
                        outt[2 * u + 1] = outt[2 * u + 1] + wgt[p] * vo
                for t in range(4):
                    out_v[i, pl.ds(16 * t, 16)] = outt[t]
            return carry2

        lax.fori_loop(0, CH // 4, q_body, 0)
        pltpu.sync_copy(out_v, out_hbm.at[b, pl.ds(c * CH, CH), h])

    buf_a = (idx_a, kv_a, q_a, w1_a, lg_a, sem_a)
    buf_b = (idx_b, kv_b, q_b, w1_b, lg_b, sem_b)

    def fire_buf(c, buf):
        fire(c, buf[0], buf[1], buf[2], buf[3], buf[4], buf[5])

    def comp_buf(c, buf):
        drain(buf[0], buf[1], buf[5])
        comp(c, buf[1], buf[2], buf[3], buf[4])

    def seq_body(c, carry):
        fire_buf(c, buf_a)
        comp_buf(c, buf_a)
        return carry

    lax.fori_loop(0, NCHUNK, seq_body, 0)


@functools.lru_cache(maxsize=1)
def _build_sc_attend():
    return pl.kernel(
        _sc_attend_body,
        mesh=plsc.VectorSubcoreMesh(core_axis_name="c", subcore_axis_name="s"),
        compiler_params=pltpu.CompilerParams(needs_layout_passes=False),
        out_type=jax.ShapeDtypeStruct((B_SZ, L_SEQ, H, D), jnp.float32),
        scratch_types=[
            pltpu.VMEM((ROWS,), jnp.int32),
            pltpu.VMEM((ROWS,), jnp.int32),
            pltpu.VMEM((ROWS, 2 * D), jnp.int32),
            pltpu.VMEM((ROWS, 2 * D), jnp.int32),
            pltpu.VMEM((CH, D), jnp.float32),
            pltpu.VMEM((CH, D), jnp.float32),
            pltpu.VMEM((CH * P,), jnp.float32),
            pltpu.VMEM((CH * P,), jnp.float32),
            pltpu.VMEM((CH * P,), jnp.float32),
            pltpu.VMEM((CH * P,), jnp.float32),
            pltpu.VMEM((CH, D), jnp.float32),
            pltpu.SemaphoreType.DMA,
            pltpu.SemaphoreType.DMA,
        ],
    )


def kernel(q_in, kv_in, Wq, bq, Wk, bk, Wv, bv, Woff, boff, Waw, baw,
           Wout, bout):
    B, L, dm = q_in.shape
    perm = jnp.asarray(_PERM)
    Wk_p = Wk[perm, :]
    Wv_p = Wv[perm, :]
    bk_p = bk[perm]
    bv_p = bv[perm]

    xq = q_in.reshape(B * L, dm)
    xkv = kv_in.reshape(B * L, dm)
    q2, k2, v2, g0, w1, lg = _proj_call(
        xq, xkv, Wq, bq, Wk_p, bk_p, Wv_p, bv_p, Woff, boff, Waw, baw)

    q4 = q2.reshape(B, L, H, D)
    kv4 = jnp.concatenate(
        [k2.reshape(B, L, H, D), v2.reshape(B, L, H, D)],
        axis=-1).transpose(0, 2, 1, 3)
    kv_shift = jnp.concatenate([kv4[:, :, 1:, :], kv4[:, :, -1:, :]], axis=2)
    kv_pair = jnp.concatenate([kv4, kv_shift], axis=-1)
    kv_flat = lax.bitcast_convert_type(
        kv_pair.reshape(B * H * L, 2 * D, 2), jnp.int32)
    idxp = g0.reshape(B, L, H, P).transpose(0, 2, 1, 3).reshape(B * H, L * P)
    w1s = w1.reshape(B, L, H, P).transpose(0, 2, 1, 3).reshape(B * H, L * P)
    lgs = lg.reshape(B, L, H, P).transpose(0, 2, 1, 3).reshape(B * H, L * P)

    ctx = _build_sc_attend()(kv_flat, q4, idxp, w1s, lgs)
    out = _outproj_call(ctx.reshape(B * L, dm), Wout, bout)
    return out.reshape(B, L, dm)


# speedup vs baseline: 1.9744x; 1.9744x over previous
import functools
import math

import jax
import jax.numpy as jnp
from jax import lax
from jax.experimental import pallas as pl
from jax.experimental.pallas import tpu as pltpu
from jax.experimental.pallas import tpu_sc as plsc

DM = 1024
H = 16
P = 4
D = DM // H
HP = H * P
L_SEQ = 2048
B_SZ = 2

BLKA = 512
CH = 64
ROWS = CH * P
NCHUNK = L_SEQ // CH

_NC = 2
_NS = 16


def _proj_body(xq, wq, bq, woff, boff, waw, baw,
               q_o, g0_o, w1_o, lg_o):
    i = pl.program_id(0)
    x = xq[...]

    def mm(a, w):
        return lax.dot_general(a, w, (((1,), (1,)), ((), ())),
                               preferred_element_type=jnp.float32)

    q_o[...] = (mm(x, wq[...]) + bq[...]) * (1.0 / math.sqrt(D))
    off = mm(x, woff[...]) + boff[...]
    lg_o[...] = mm(x, waw[...]) + baw[...]

    rows = i * BLKA + lax.broadcasted_iota(jnp.int32, (BLKA, 1), 0)
    lpos = lax.rem(rows, L_SEQ)
    bidx = rows // L_SEQ
    idx_f = jnp.clip(lpos.astype(jnp.float32) + off, 0.0, float(L_SEQ - 1))
    i0f = jnp.floor(idx_f)
    w1_o[...] = idx_f - i0f
    i0 = i0f.astype(jnp.int32)
    hcol = lax.broadcasted_iota(jnp.int32, (1, HP), 1) // P
    bhoff = (bidx * H + hcol) * L_SEQ
    g0_o[...] = bhoff + i0


def _proj_call(xq, Wq, bq, Woff, boff, Waw, baw):
    n = xq.shape[0]
    row_spec = pl.BlockSpec((BLKA, DM), lambda i: (i, 0))
    hp_spec = pl.BlockSpec((BLKA, HP), lambda i: (i, 0))
    full = lambda shape: pl.BlockSpec(shape, lambda i: tuple(0 for _ in shape))
    return pl.pallas_call(
        _proj_body,
        grid=(n // BLKA,),
        in_specs=[
            row_spec,
            full((DM, DM)), full((1, DM)),
            full((HP, DM)), full((1, HP)),
            full((HP, DM)), full((1, HP)),
        ],
        out_specs=[row_spec, hp_spec, hp_spec, hp_spec],
        out_shape=[
            jax.ShapeDtypeStruct((n, DM), jnp.float32),
            jax.ShapeDtypeStruct((n, HP), jnp.int32),
            jax.ShapeDtypeStruct((n, HP), jnp.float32),
            jax.ShapeDtypeStruct((n, HP), jnp.float32),
        ],
    )(xq, Wq, bq.reshape(1, DM), Woff, boff.reshape(1, HP),
      Waw, baw.reshape(1, HP))


def _kvpack_body(x_ref, wkv_ref, bkv_ref, o_ref):
    x = x_ref[0]
    w = wkv_ref[0]
    kv = lax.dot_general(x, w, (((1,), (0,)), ((), ())),
                         preferred_element_type=jnp.float32)
    kv = kv + bkv_ref[0]
    kvb = kv.astype(jnp.bfloat16).astype(jnp.float32)
    bits = lax.bitcast_convert_type(kvb, jnp.int32)
    lo_k, hi_k = bits[:, 0:32], bits[:, 32:64]
    lo_v, hi_v = bits[:, 64:96], bits[:, 96:128]
    mask_hi = jnp.int32(-65536)
    kw = jnp.bitwise_or(lax.shift_right_logical(lo_k, 16),
                        jnp.bitwise_and(hi_k, mask_hi))
    vw = jnp.bitwise_or(lax.shift_right_logical(lo_v, 16),
                        jnp.bitwise_and(hi_v, mask_hi))
    row = jnp.concatenate([kw, vw], axis=1)
    row_next = jnp.concatenate([row[1:, :], row[L_SEQ - 1:, :]], axis=0)
    o_ref[...] = jnp.concatenate([row, row_next], axis=1)[None, None]


def _kvpack_call(xkv3, WKV, bKV):
    return pl.pallas_call(
        _kvpack_body,
        grid=(B_SZ, H),
        in_specs=[
            pl.BlockSpec((1, L_SEQ, DM), lambda b, h: (b, 0, 0)),
            pl.BlockSpec((1, DM, 2 * D), lambda b, h: (h, 0, 0)),
            pl.BlockSpec((1, 1, 2 * D), lambda b, h: (h, 0, 0)),
        ],
        out_specs=pl.BlockSpec((1, 1, L_SEQ, 4 * D // 2),
                               lambda b, h: (b, h, 0, 0)),
        out_shape=jax.ShapeDtypeStruct((B_SZ, H, L_SEQ, 2 * D), jnp.int32),
    )(xkv3, WKV, bKV)


def _outproj_body(x_ref, w_ref, b_ref, o_ref):
    o_ref[...] = lax.dot_general(
        x_ref[...], w_ref[...], (((1,), (1,)), ((), ())),
        preferred_element_type=jnp.float32) + b_ref[...]


def _outproj_call(x, Wout, bout):
    n = x.shape[0]
    row_spec = pl.BlockSpec((BLKA, DM), lambda i: (i, 0))
    return pl.pallas_call(
        _outproj_body,
        grid=(n // BLKA,),
        in_specs=[row_spec,
                  pl.BlockSpec((DM, DM), lambda i: (0, 0)),
                  pl.BlockSpec((1, DM), lambda i: (0, 0))],
        out_specs=row_spec,
        out_shape=jax.ShapeDtypeStruct((n, DM), jnp.float32),
    )(x, Wout, bout.reshape(1, DM))


def _sc_attend_body(kv_hbm, q_hbm, idx_hbm, w1_hbm, lg_hbm, out_hbm,
                    idx_a, idx_b, kv_a, kv_b, q_a, q_b, w1_a, w1_b,
                    lg_a, lg_b, out_v, sem_a, sem_b):
    cid = lax.axis_index("c")
    sid = lax.axis_index("s")
    bh = sid * _NC + cid
    b = bh // H
    h = lax.rem(bh, H)

    NGATHER = ROWS // 128

    def fire(c, idx_v, kv_v, q_v, w1_v, lg_v, sem):
        pltpu.sync_copy(idx_hbm.at[bh, pl.ds(c * ROWS, ROWS)], idx_v)
        for j in range(NGATHER):
            pltpu.async_copy(kv_hbm.at[idx_v.at[pl.ds(j * 128, 128)]],
                             kv_v.at[pl.ds(j * 128, 128)], sem)
        pltpu.sync_copy(q_hbm.at[b, pl.ds(c * CH, CH), h], q_v)
        pltpu.sync_copy(w1_hbm.at[bh, pl.ds(c * CH * P, CH * P)], w1_v)
        pltpu.sync_copy(lg_hbm.at[bh, pl.ds(c * CH * P, CH * P)], lg_v)

    def drain(idx_v, kv_v, sem):
        for j in range(NGATHER):
            pltpu.make_async_copy(
                kv_hbm.at[idx_v.at[pl.ds(j * 128, 128)]],
                kv_v.at[pl.ds(j * 128, 128)], sem).wait()

    def interp_pair(kv_v, r, off0, off1, w1p):
        x0 = plsc.bitcast(kv_v[r, pl.ds(off0, 16)], jnp.bfloat16)
        x1 = plsc.bitcast(kv_v[r, pl.ds(off1, 16)], jnp.bfloat16)
        e0, o0 = plsc.unpack(x0, format=plsc.PackFormat.INTERLEAVED)
        e1, o1 = plsc.unpack(x1, format=plsc.PackFormat.INTERLEAVED)
        ee = e0 + w1p * (e1 - e0)
        oo = o0 + w1p * (o1 - o0)
        return ee, oo

    def comp(c, kv_v, q_v, w1_v, lg_v):
        def q_body(g, carry2):
            wvec = w1_v[pl.ds(g * 16, 16)]
            lvec = lg_v[pl.ds(g * 16, 16)]
            for qq in range(4):
                i = g * 4 + qq
                qi = [q_v[i, pl.ds(16 * t, 16)] for t in range(4)]
                base = i * P
                w1ps = []
                scrs = []
                for p in range(P):
                    w1p = jnp.full((16,), wvec[4 * qq + p])
                    lgp = lvec[4 * qq + p]
                    w1ps.append(w1p)
                    r = base + p
                    ke0, ko0 = interp_pair(kv_v, r, 0, 64, w1p)
                    ke1, ko1 = interp_pair(kv_v, r, 16, 80, w1p)
                    acc = ((qi[0] * ke0 + qi[2] * ko0)
                           + (qi[1] * ke1 + qi[3] * ko1))
                    scrs.append(lgp + jnp.sum(acc))
                m = jnp.maximum(jnp.maximum(scrs[0], scrs[1]),
                                jnp.maximum(scrs[2], scrs[3]))
                es = [jnp.exp(jnp.full((16,), s - m)) for s in scrs]
                den = (es[0] + es[1]) + (es[2] + es[3])
                wgt = [e / den for e in es]
                outt = [jnp.zeros((16,), jnp.float32) for _ in range(4)]
                for p in range(P):
                    r = base + p
                    for u in range(2):
                        ve, vo = interp_pair(kv_v, r, 32 + 16 * u,
                                             96 + 16 * u, w1ps[p])
                        outt[u] = outt[u] + wgt[p] * ve
                        outt[2 + u] = outt[2 + u] + wgt[p] * vo
                for t in range(4):
                    out_v[i, pl.ds(16 * t, 16)] = outt[t]
            return carry2

        lax.fori_loop(0, CH // 4, q_body, 0)
        pltpu.sync_copy(out_v, out_hbm.at[b, pl.ds(c * CH, CH), h])

    buf_a = (idx_a, kv_a, q_a, w1_a, lg_a, sem_a)
    buf_b = (idx_b, kv_b, q_b, w1_b, lg_b, sem_b)

    def fire_buf(c, buf):
        fire(c, buf[0], buf[1], buf[2], buf[3], buf[4], buf[5])

    def comp_buf(c, buf):
        drain(buf[0], buf[1], buf[5])
        comp(c, buf[1], buf[2], buf[3], buf[4])

    fire_buf(0, buf_a)

    def pipe_body(it, carry):
        c0 = it * 2
        fire_buf(c0 + 1, buf_b)
        comp_buf(c0, buf_a)
        fire_buf(c0 + 2, buf_a)
        comp_buf(c0 + 1, buf_b)
        return carry

    lax.fori_loop(0, (NCHUNK - 2) // 2, pipe_body, 0)
    fire_buf(NCHUNK - 1, buf_b)
    comp_buf(NCHUNK - 2, buf_a)
    comp_buf(NCHUNK - 1, buf_b)


@functools.lru_cache(maxsize=1)
def _build_sc_attend():
    return pl.kernel(
        _sc_attend_body,
        mesh=plsc.VectorSubcoreMesh(core_axis_name="c", subcore_axis_name="s"),
        compiler_params=pltpu.CompilerParams(needs_layout_passes=False),
        out_type=jax.ShapeDtypeStruct((B_SZ, L_SEQ, H, D), jnp.float32),
        scratch_types=[
            pltpu.VMEM((ROWS,), jnp.int32),
            pltpu.VMEM((ROWS,), jnp.int32),
            pltpu.VMEM((ROWS, 2 * D), jnp.int32),
            pltpu.VMEM((ROWS, 2 * D), jnp.int32),
            pltpu.VMEM((CH, D), jnp.float32),
            pltpu.VMEM((CH, D), jnp.float32),
            pltpu.VMEM((CH * P,), jnp.float32),
            pltpu.VMEM((CH * P,), jnp.float32),
            pltpu.VMEM((CH * P,), jnp.float32),
            pltpu.VMEM((CH * P,), jnp.float32),
            pltpu.VMEM((CH, D), jnp.float32),
            pltpu.SemaphoreType.DMA,
            pltpu.SemaphoreType.DMA,
        ],
    )


def kernel(q_in, kv_in, Wq, bq, Wk, bk, Wv, bv, Woff, boff, Waw, baw,
           Wout, bout):
    B, L, dm = q_in.shape
    xq = q_in.reshape(B * L, dm)
    q2, g0, w1, lg = _proj_call(xq, Wq, bq, Woff, boff, Waw, baw)
    q4 = q2.reshape(B, L, H, D)

    WKV = jnp.concatenate(
        [Wk.T.reshape(dm, H, D), Wv.T.reshape(dm, H, D)],
        axis=-1).transpose(1, 0, 2)
    bKV = jnp.concatenate(
        [bk.reshape(H, 1, D), bv.reshape(H, 1, D)], axis=-1)
    kv_pack = _kvpack_call(kv_in, WKV, bKV)
    kv_flat = kv_pack.reshape(B * H * L, 2 * D)

    idxp = g0.reshape(B, L, H, P).transpose(0, 2, 1, 3).reshape(B * H, L * P)
    w1s = w1.reshape(B, L, H, P).transpose(0, 2, 1, 3).reshape(B * H, L * P)
    lgs = lg.reshape(B, L, H, P).transpose(0, 2, 1, 3).reshape(B * H, L * P)

    ctx = _build_sc_attend()(kv_flat, q4, idxp, w1s, lgs)
    out = _outproj_call(ctx.reshape(B * L, dm), Wout, bout)
    return out.reshape(B, L, dm)


# speedup vs baseline: 1.9746x; 1.0001x over previous
import functools
import math

import jax
import jax.numpy as jnp
from jax import lax
from jax.experimental import pallas as pl
from jax.experimental.pallas import tpu as pltpu
from jax.experimental.pallas import tpu_sc as plsc

DM = 1024
H = 16
P = 4
D = DM // H
HP = H * P
L_SEQ = 2048
B_SZ = 2

BLKA = 512
CH = 64
ROWS = CH * P
NCHUNK = L_SEQ // CH

_NC = 2
_NS = 16


def _proj_body(xq, wq, bq, woff, boff, waw, baw,
               q_o, g0_o, w1_o, lg_o):
    i = pl.program_id(0)
    x = xq[...]

    def mm(a, w):
        return lax.dot_general(a.astype(jnp.bfloat16),
                               w.astype(jnp.bfloat16),
                               (((1,), (1,)), ((), ())),
                               preferred_element_type=jnp.float32)

    q_o[...] = (mm(x, wq[...]) + bq[...]) * (1.0 / math.sqrt(D))
    off = mm(x, woff[...]) + boff[...]
    lg_o[...] = mm(x, waw[...]) + baw[...]

    rows = i * BLKA + lax.broadcasted_iota(jnp.int32, (BLKA, 1), 0)
    lpos = lax.rem(rows, L_SEQ)
    bidx = rows // L_SEQ
    idx_f = jnp.clip(lpos.astype(jnp.float32) + off, 0.0, float(L_SEQ - 1))
    i0f = jnp.floor(idx_f)
    w1_o[...] = idx_f - i0f
    i0 = i0f.astype(jnp.int32)
    hcol = lax.broadcasted_iota(jnp.int32, (1, HP), 1) // P
    bhoff = (bidx * H + hcol) * L_SEQ
    g0_o[...] = bhoff + i0


def _proj_call(xq, Wq, bq, Woff, boff, Waw, baw):
    n = xq.shape[0]
    row_spec = pl.BlockSpec((BLKA, DM), lambda i: (i, 0))
    hp_spec = pl.BlockSpec((BLKA, HP), lambda i: (i, 0))
    full = lambda shape: pl.BlockSpec(shape, lambda i: tuple(0 for _ in shape))
    return pl.pallas_call(
        _proj_body,
        grid=(n // BLKA,),
        in_specs=[
            row_spec,
            full((DM, DM)), full((1, DM)),
            full((HP, DM)), full((1, HP)),
            full((HP, DM)), full((1, HP)),
        ],
        out_specs=[row_spec, hp_spec, hp_spec, hp_spec],
        out_shape=[
            jax.ShapeDtypeStruct((n, DM), jnp.float32),
            jax.ShapeDtypeStruct((n, HP), jnp.int32),
            jax.ShapeDtypeStruct((n, HP), jnp.float32),
            jax.ShapeDtypeStruct((n, HP), jnp.float32),
        ],
    )(xq, Wq, bq.reshape(1, DM), Woff, boff.reshape(1, HP),
      Waw, baw.reshape(1, HP))


def _kvpack_body(x_ref, wkv_ref, bkv_ref, o_ref):
    x = x_ref[0]
    w = wkv_ref[0]
    kv = lax.dot_general(x.astype(jnp.bfloat16), w.astype(jnp.bfloat16),
                         (((1,), (0,)), ((), ())),
                         preferred_element_type=jnp.float32)
    kv = kv + bkv_ref[0]
    kvb = kv.astype(jnp.bfloat16).astype(jnp.float32)
    bits = lax.bitcast_convert_type(kvb, jnp.int32)
    lo_k, hi_k = bits[:, 0:32], bits[:, 32:64]
    lo_v, hi_v = bits[:, 64:96], bits[:, 96:128]
    mask_hi = jnp.int32(-65536)
    kw = jnp.bitwise_or(lax.shift_right_logical(lo_k, 16),
                        jnp.bitwise_and(hi_k, mask_hi))
    vw = jnp.bitwise_or(lax.shift_right_logical(lo_v, 16),
                        jnp.bitwise_and(hi_v, mask_hi))
    row = jnp.concatenate([kw, vw], axis=1)
    row_next = jnp.concatenate([row[1:, :], row[L_SEQ - 1:, :]], axis=0)
    o_ref[...] = jnp.concatenate([row, row_next], axis=1)[None, None]


def _kvpack_call(xkv3, WKV, bKV):
    return pl.pallas_call(
        _kvpack_body,
        grid=(B_SZ, H),
        in_specs=[
            pl.BlockSpec((1, L_SEQ, DM), lambda b, h: (b, 0, 0)),
            pl.BlockSpec((1, DM, 2 * D), lambda b, h: (h, 0, 0)),
            pl.BlockSpec((1, 1, 2 * D), lambda b, h: (h, 0, 0)),
        ],
        out_specs=pl.BlockSpec((1, 1, L_SEQ, 4 * D // 2),
                               lambda b, h: (b, h, 0, 0)),
        out_shape=jax.ShapeDtypeStruct((B_SZ, H, L_SEQ, 2 * D), jnp.int32),
    )(xkv3, WKV, bKV)


def _outproj_body(x_ref, w_ref, b_ref, o_ref):
    o_ref[...] = lax.dot_general(
        x_ref[...].astype(jnp.bfloat16), w_ref[...].astype(jnp.bfloat16),
        (((1,), (1,)), ((), ())),
        preferred_element_type=jnp.float32) + b_ref[...]


def _outproj_call(x, Wout, bout):
    n = x.shape[0]
    row_spec = pl.BlockSpec((BLKA, DM), lambda i: (i, 0))
    return pl.pallas_call(
        _outproj_body,
        grid=(n // BLKA,),
        in_specs=[row_spec,
                  pl.BlockSpec((DM, DM), lambda i: (0, 0)),
                  pl.BlockSpec((1, DM), lambda i: (0, 0))],
        out_specs=row_spec,
        out_shape=jax.ShapeDtypeStruct((n, DM), jnp.float32),
    )(x, Wout, bout.reshape(1, DM))


def _sc_attend_body(kv_hbm, q_hbm, idx_hbm, w1_hbm, lg_hbm, out_hbm,
                    idx_a, idx_b, kv_a, kv_b, q_a, q_b, w1_a, w1_b,
                    lg_a, lg_b, out_v, sem_a, sem_b):
    cid = lax.axis_index("c")
    sid = lax.axis_index("s")
    bh = sid * _NC + cid
    b = bh // H
    h = lax.rem(bh, H)

    NGATHER = ROWS // 128

    def fire(c, idx_v, kv_v, q_v, w1_v, lg_v, sem):
        pltpu.sync_copy(idx_hbm.at[bh, pl.ds(c * ROWS, ROWS)], idx_v)
        for j in range(NGATHER):
            pltpu.async_copy(kv_hbm.at[idx_v.at[pl.ds(j * 128, 128)]],
                             kv_v.at[pl.ds(j * 128, 128)], sem)
        pltpu.sync_copy(q_hbm.at[b, pl.ds(c * CH, CH), h], q_v)
        pltpu.sync_copy(w1_hbm.at[bh, pl.ds(c * CH * P, CH * P)], w1_v)
        pltpu.sync_copy(lg_hbm.at[bh, pl.ds(c * CH * P, CH * P)], lg_v)

    def drain(idx_v, kv_v, sem):
        for j in range(NGATHER):
            pltpu.make_async_copy(
                kv_hbm.at[idx_v.at[pl.ds(j * 128, 128)]],
                kv_v.at[pl.ds(j * 128, 128)], sem).wait()

    def interp_pair(kv_v, r, off0, off1, w1p):
        x0 = plsc.bitcast(kv_v[r, pl.ds(off0, 16)], jnp.bfloat16)
        x1 = plsc.bitcast(kv_v[r, pl.ds(off1, 16)], jnp.bfloat16)
        e0, o0 = plsc.unpack(x0, format=plsc.PackFormat.INTERLEAVED)
        e1, o1 = plsc.unpack(x1, format=plsc.PackFormat.INTERLEAVED)
        ee = e0 + w1p * (e1 - e0)
        oo = o0 + w1p * (o1 - o0)
        return ee, oo

    def comp(c, kv_v, q_v, w1_v, lg_v):
        def q_body(g, carry2):
            wvec = w1_v[pl.ds(g * 16, 16)]
            lvec = lg_v[pl.ds(g * 16, 16)]
            for qq in range(4):
                i = g * 4 + qq
                qi = [q_v[i, pl.ds(16 * t, 16)] for t in range(4)]
                base = i * P
                w1ps = []
                scrs = []
                for p in range(P):
                    w1p = jnp.full((16,), wvec[4 * qq + p])
                    lgp = lvec[4 * qq + p]
                    w1ps.append(w1p)
                    r = base + p
                    ke0, ko0 = interp_pair(kv_v, r, 0, 64, w1p)
                    ke1, ko1 = interp_pair(kv_v, r, 16, 80, w1p)
                    acc = ((qi[0] * ke0 + qi[2] * ko0)
                           + (qi[1] * ke1 + qi[3] * ko1))
                    scrs.append(lgp + jnp.sum(acc))
                m = jnp.maximum(jnp.maximum(scrs[0], scrs[1]),
                                jnp.maximum(scrs[2], scrs[3]))
                es = [jnp.exp(jnp.full((16,), s - m)) for s in scrs]
                den = (es[0] + es[1]) + (es[2] + es[3])
                wgt = [e / den for e in es]
                outt = [jnp.zeros((16,), jnp.float32) for _ in range(4)]
                for p in range(P):
                    r = base + p
                    for u in range(2):
                        ve, vo = interp_pair(kv_v, r, 32 + 16 * u,
                                             96 + 16 * u, w1ps[p])
                        outt[u] = outt[u] + wgt[p] * ve
                        outt[2 + u] = outt[2 + u] + wgt[p] * vo
                for t in range(4):
                    out_v[i, pl.ds(16 * t, 16)] = outt[t]
            return carry2

        lax.fori_loop(0, CH // 4, q_body, 0)
        pltpu.sync_copy(out_v, out_hbm.at[b, pl.ds(c * CH, CH), h])

    buf_a = (idx_a, kv_a, q_a, w1_a, lg_a, sem_a)
    buf_b = (idx_b, kv_b, q_b, w1_b, lg_b, sem_b)

    def fire_buf(c, buf):
        fire(c, buf[0], buf[1], buf[2], buf[3], buf[4], buf[5])

    def comp_buf(c, buf):
        drain(buf[0], buf[1], buf[5])
        comp(c, buf[1], buf[2], buf[3], buf[4])

    fire_buf(0, buf_a)

    def pipe_body(it, carry):
        c0 = it * 2
        fire_buf(c0 + 1, buf_b)
        comp_buf(c0, buf_a)
        fire_buf(c0 + 2, buf_a)
        comp_buf(c0 + 1, buf_b)
        return carry

    lax.fori_loop(0, (NCHUNK - 2) // 2, pipe_body, 0)
    fire_buf(NCHUNK - 1, buf_b)
    comp_buf(NCHUNK - 2, buf_a)
    comp_buf(NCHUNK - 1, buf_b)


@functools.lru_cache(maxsize=1)
def _build_sc_attend():
    return pl.kernel(
        _sc_attend_body,
        mesh=plsc.VectorSubcoreMesh(core_axis_name="c", subcore_axis_name="s"),
        compiler_params=pltpu.CompilerParams(needs_layout_passes=False),
        out_type=jax.ShapeDtypeStruct((B_SZ, L_SEQ, H, D), jnp.float32),
        scratch_types=[
            pltpu.VMEM((ROWS,), jnp.int32),
            pltpu.VMEM((ROWS,), jnp.int32),
            pltpu.VMEM((ROWS, 2 * D), jnp.int32),
            pltpu.VMEM((ROWS, 2 * D), jnp.int32),
            pltpu.VMEM((CH, D), jnp.float32),
            pltpu.VMEM((CH, D), jnp.float32),
            pltpu.VMEM((CH * P,), jnp.float32),
            pltpu.VMEM((CH * P,), jnp.float32),
            pltpu.VMEM((CH * P,), jnp.float32),
            pltpu.VMEM((CH * P,), jnp.float32),
            pltpu.VMEM((CH, D), jnp.float32),
            pltpu.SemaphoreType.DMA,
            pltpu.SemaphoreType.DMA,
        ],
    )


def kernel(q_in, kv_in, Wq, bq, Wk, bk, Wv, bv, Woff, boff, Waw, baw,
           Wout, bout):
    B, L, dm = q_in.shape
    xq = q_in.reshape(B * L, dm)
    q2, g0, w1, lg = _proj_call(xq, Wq, bq, Woff, boff, Waw, baw)
    q4 = q2.reshape(B, L, H, D)

    WKV = jnp.concatenate(
        [Wk.T.reshape(dm, H, D), Wv.T.reshape(dm, H, D)],
        axis=-1).transpose(1, 0, 2)
    bKV = jnp.concatenate(
        [bk.reshape(H, 1, D), bv.reshape(H, 1, D)], axis=-1)
    kv_pack = _kvpack_call(kv_in, WKV, bKV)
    kv_flat = kv_pack.reshape(B * H * L, 2 * D)

    idxp = g0.reshape(B, L, H, P).transpose(0, 2, 1, 3).reshape(B * H, L * P)
    w1s = w1.reshape(B, L, H, P).transpose(0, 2, 1, 3).reshape(B * H, L * P)
    lgs = lg.reshape(B, L, H, P).transpose(0, 2, 1, 3).reshape(B * H, L * P)

    ctx = _build_sc_attend()(kv_flat, q4, idxp, w1s, lgs)
    out = _outproj_call(ctx.reshape(B * L, dm), Wout, bout)
    return out.reshape(B, L, dm)
